# baseline (speedup 1.0000x reference)
import dataclasses
import functools

import jax
import jax.numpy as jnp
from jax import lax
from jax.experimental import pallas as pl
from jax.experimental.pallas import tpu as pltpu
from jax.experimental.pallas import tpu_sc as plsc

N = 10000
E = 160000
D = 256
DH = 128
NPAD = 10240
NC = 2
NS = 16
EPS = E // NS
CH = 128
NCH = EPS // CH
TAIL = EPS - NCH * CH
ZR = 128
RPS = NPAD // NS


_RB = 400
_NRB = N // _RB


def _proj_body(h_ref, w2_ref, out_ref):
    out_ref[0] = lax.dot_general(
        h_ref[...], w2_ref[...],
        (((1,), (1,)), ((), ())),
        preferred_element_type=jnp.float32,
    )


def _proj(h, W2):
    return pl.pallas_call(
        _proj_body,
        grid=(_NRB, NC),
        in_specs=[
            pl.BlockSpec((_RB, D), lambda i, j: (i, 0)),
            pl.BlockSpec((DH, D), lambda i, j: (j, 0)),
        ],
        out_specs=pl.BlockSpec((1, _RB, DH), lambda i, j: (j, i, 0)),
        out_shape=jax.ShapeDtypeStruct((NC, NPAD, DH), jnp.float32),
    )(h, W2)


def _sc_agg(p2, src, dst):
    mesh = plsc.VectorSubcoreMesh(core_axis_name="c", subcore_axis_name="s")
    cp = pltpu.CompilerParams()
    if "needs_layout_passes" in pltpu.CompilerParams.__dataclass_fields__:
        cp = dataclasses.replace(cp, needs_layout_passes=False)

    @functools.partial(
        pl.kernel,
        compiler_params=cp,
        out_type=(
            jax.ShapeDtypeStruct((NC * NPAD, DH), jnp.float32),
            jax.ShapeDtypeStruct((NS, NPAD), jnp.float32),
        ),
        mesh=mesh,
        scratch_types=[
            pltpu.VMEM((2, CH), jnp.int32),
            pltpu.VMEM((2, CH), jnp.int32),
            pltpu.VMEM((2, CH, DH), jnp.float32),
            pltpu.VMEM((TAIL,), jnp.int32),
            pltpu.VMEM((TAIL,), jnp.int32),
            pltpu.VMEM((TAIL, DH), jnp.float32),
            pltpu.VMEM((NPAD,), jnp.float32),
            pltpu.VMEM_SHARED((NPAD, DH), jnp.float32),
            pltpu.SemaphoreType.DMA,
            pltpu.SemaphoreType.DMA,
        ],
    )
    def k(p2_h, src_h, dst_h, acc_h, deg_h,
          isrc, idst, rows, tsrc, tdst, trows, hist,
          acc_sh, sg0, sg1):
        c = lax.axis_index("c")
        s = lax.axis_index("s")
        core0 = c == 0
        off = c * NPAD

        zv = jnp.zeros((16,), jnp.float32)
        ov = jnp.ones((16,), jnp.float32)

        @pl.loop(0, ZR)
        def _(i):
            @pl.loop(0, DH, step=16)
            def _(j):
                rows[0, i, pl.ds(j, 16)] = zv

        @pl.loop(0, NPAD, step=16)
        def _(i):
            hist[pl.ds(i, 16)] = zv

        rbase = s * RPS

        @pl.loop(0, RPS, step=ZR)
        def _(r):
            pltpu.sync_copy(rows.at[0], acc_sh.at[pl.ds(rbase + r, ZR)])

        plsc.subcore_barrier()

        ebase = s * EPS

        def load_idx(k_, b):
            pltpu.sync_copy(src_h.at[pl.ds(ebase + k_ * CH, CH)], isrc.at[b])
            pltpu.sync_copy(dst_h.at[pl.ds(ebase + k_ * CH, CH)], idst.at[b])

            @pl.loop(0, CH, step=16)
            def _(j):
                isrc[b, pl.ds(j, 16)] = isrc[b, pl.ds(j, 16)] + off

        def scatter(b):
            pltpu.sync_copy(rows.at[b], acc_sh.at[idst.at[b]], add=True)

            @pl.when(core0)
            def _():
                @pl.loop(0, CH, step=16)
                def _(j):
                    plsc.addupdate_scatter(hist, [idst[b, pl.ds(j, 16)]], ov)

        load_idx(0, 0)
        load_idx(1, 1)

        @pl.loop(0, NCH, step=2)
        def _(k_):
            d0 = pltpu.async_copy(p2_h.at[isrc.at[0]], rows.at[0], sg0)
            d1 = pltpu.async_copy(p2_h.at[isrc.at[1]], rows.at[1], sg1)
            d0.wait()
            scatter(0)
            load_idx(k_ + 2, 0)
            d1.wait()
            scatter(1)
            load_idx(k_ + 3, 1)

        tbase = ebase + NCH * CH
        pltpu.sync_copy(src_h.at[pl.ds(tbase, TAIL)], tsrc)
        pltpu.sync_copy(dst_h.at[pl.ds(tbase, TAIL)], tdst)
        tsrc[...] = tsrc[...] + off
        pltpu.sync_copy(p2_h.at[tsrc], trows)
        pltpu.sync_copy(trows, acc_sh.at[tdst], add=True)

        @pl.when(core0)
        def _():
            plsc.addupdate_scatter(hist, [tdst[...]], ov)

        plsc.subcore_barrier()

        pltpu.sync_copy(acc_sh.at[pl.ds(rbase, RPS)],
                        acc_h.at[pl.ds(off + rbase, RPS)])

        @pl.when(core0)
        def _():
            pltpu.sync_copy(hist, deg_h.at[s])

    return k(p2, src, dst)


def _combine_body(h_ref, w1_ref, b2_ref, acc0_ref, acc1_ref, deg_ref, out_ref):
    q = lax.dot_general(
        h_ref[...], w1_ref[...],
        (((1,), (1,)), ((), ())),
        preferred_element_type=jnp.float32,
    ) + b2_ref[...]
    deg = jnp.sum(deg_ref[0], axis=0)[:, None]
    inv = 1.0 / jnp.maximum(deg, 1.0)
    hn = jnp.concatenate([acc0_ref[0], acc1_ref[0]], axis=1) * inv
    t = q + hn
    ss = jnp.sum(t * t, axis=1, keepdims=True)
    out_ref[...] = t / jnp.maximum(jnp.sqrt(ss), 1e-12)


def _combine(h, W1, b2, acc, deg):
    return pl.pallas_call(
        _combine_body,
        grid=(_NRB,),
        in_specs=[
            pl.BlockSpec((_RB, D), lambda i: (i, 0)),
            pl.BlockSpec((D, D), lambda i: (0, 0)),
            pl.BlockSpec((1, D), lambda i: (0, 0)),
            pl.BlockSpec((1, _RB, DH), lambda i: (0, i, 0)),
            pl.BlockSpec((1, _RB, DH), lambda i: (1, i, 0)),
            pl.BlockSpec((1, NS, _RB), lambda i: (i, 0, 0)),
        ],
        out_specs=pl.BlockSpec((_RB, D), lambda i: (i, 0)),
        out_shape=jax.ShapeDtypeStruct((N, D), jnp.float32),
    )(h, W1, b2, acc, acc, deg)


def kernel(h, edge_index, W1, W2, b2):
    zpad = jnp.zeros((2, 2 * CH), jnp.int32)
    eip = jnp.concatenate([edge_index, zpad], axis=1)
    src = eip[0]
    dst = eip[1]
    p2 = _proj(h, W2)
    acc, deg = _sc_agg(p2.reshape(NC * NPAD, DH), src, dst)
    deg3 = deg[:, :N].reshape(NS, _NRB, _RB).transpose(1, 0, 2)
    out = _combine(h, W1, b2.reshape(1, D), acc.reshape(NC, NPAD, DH), deg3)
    return out


# speedup vs baseline: 1.2109x; 1.2109x over previous
import dataclasses
import functools

import jax
import jax.numpy as jnp
from jax import lax
from jax.experimental import pallas as pl
from jax.experimental.pallas import tpu as pltpu
from jax.experimental.pallas import tpu_sc as plsc

N = 10000
E = 160000
D = 256
DH = 128
NPAD = 10240
NC = 2
NS = 16
EPS = E // NS
CH = 128
NCH = EPS // CH
TAIL = EPS - NCH * CH
ZR = 128
RPS = NPAD // NS


_RB = 400
_NRB = N // _RB


def _proj_body(h_ref, w2_ref, out_ref):
    out_ref[0] = lax.dot_general(
        h_ref[...], w2_ref[...],
        (((1,), (1,)), ((), ())),
        preferred_element_type=jnp.float32,
    )


def _proj(h, W2):
    return pl.pallas_call(
        _proj_body,
        grid=(_NRB, NC),
        in_specs=[
            pl.BlockSpec((_RB, D), lambda i, j: (i, 0)),
            pl.BlockSpec((DH, D), lambda i, j: (j, 0)),
        ],
        out_specs=pl.BlockSpec((1, _RB, DH), lambda i, j: (j, i, 0)),
        out_shape=jax.ShapeDtypeStruct((NC, NPAD, DH), jnp.float32),
    )(h, W2)


def _sc_agg(p2, src, dst):
    mesh = plsc.VectorSubcoreMesh(core_axis_name="c", subcore_axis_name="s")
    cp = pltpu.CompilerParams()
    if "needs_layout_passes" in pltpu.CompilerParams.__dataclass_fields__:
        cp = dataclasses.replace(cp, needs_layout_passes=False)

    @functools.partial(
        pl.kernel,
        compiler_params=cp,
        out_type=(
            jax.ShapeDtypeStruct((NC * NPAD, DH), jnp.float32),
            jax.ShapeDtypeStruct((NS, NPAD), jnp.float32),
        ),
        mesh=mesh,
        scratch_types=[
            pltpu.VMEM((2, CH), jnp.int32),
            pltpu.VMEM((2, CH), jnp.int32),
            pltpu.VMEM((2, CH, DH), jnp.float32),
            pltpu.VMEM((TAIL,), jnp.int32),
            pltpu.VMEM((TAIL,), jnp.int32),
            pltpu.VMEM((TAIL, DH), jnp.float32),
            pltpu.VMEM((NPAD,), jnp.float32),
            pltpu.VMEM_SHARED((NPAD, DH), jnp.float32),
            pltpu.SemaphoreType.DMA,
            pltpu.SemaphoreType.DMA,
        ],
    )
    def k(p2_h, src_h, dst_h, acc_h, deg_h,
          isrc, idst, rows, tsrc, tdst, trows, hist,
          acc_sh, sg0, sg1):
        c = lax.axis_index("c")
        s = lax.axis_index("s")
        core0 = c == 0
        off = c * NPAD

        zv = jnp.zeros((16,), jnp.float32)
        ov = jnp.ones((16,), jnp.float32)

        @pl.loop(0, ZR)
        def _(i):
            @pl.loop(0, DH, step=16)
            def _(j):
                rows[0, i, pl.ds(j, 16)] = zv

        @pl.loop(0, NPAD, step=16)
        def _(i):
            hist[pl.ds(i, 16)] = zv

        rbase = s * RPS

        @pl.loop(0, RPS, step=ZR)
        def _(r):
            pltpu.sync_copy(rows.at[0], acc_sh.at[pl.ds(rbase + r, ZR)])

        plsc.subcore_barrier()

        ebase = s * EPS

        def load_idx(k_, b):
            pltpu.sync_copy(src_h.at[pl.ds(ebase + k_ * CH, CH)], isrc.at[b])
            pltpu.sync_copy(dst_h.at[pl.ds(ebase + k_ * CH, CH)], idst.at[b])

            @pl.loop(0, CH, step=16)
            def _(j):
                isrc[b, pl.ds(j, 16)] = isrc[b, pl.ds(j, 16)] + off

        def scatter(b):
            pltpu.sync_copy(rows.at[b], acc_sh.at[idst.at[b]], add=True)

            @pl.when(core0)
            def _():
                @pl.loop(0, CH, step=16)
                def _(j):
                    plsc.addupdate_scatter(hist, [idst[b, pl.ds(j, 16)]], ov)

        @pl.loop(0, NCH, step=2)
        def _(k_):
            load_idx(k_, 0)
            d0 = pltpu.async_copy(p2_h.at[isrc.at[0]], rows.at[0], sg0)
            load_idx(k_ + 1, 1)
            d1 = pltpu.async_copy(p2_h.at[isrc.at[1]], rows.at[1], sg1)
            d0.wait()
            scatter(0)
            d1.wait()
            scatter(1)

        tbase = ebase + NCH * CH
        pltpu.sync_copy(src_h.at[pl.ds(tbase, TAIL)], tsrc)
        pltpu.sync_copy(dst_h.at[pl.ds(tbase, TAIL)], tdst)
        tsrc[...] = tsrc[...] + off
        pltpu.sync_copy(p2_h.at[tsrc], trows)
        pltpu.sync_copy(trows, acc_sh.at[tdst], add=True)

        @pl.when(core0)
        def _():
            plsc.addupdate_scatter(hist, [tdst[...]], ov)

        plsc.subcore_barrier()

        pltpu.sync_copy(acc_sh.at[pl.ds(rbase, RPS)],
                        acc_h.at[pl.ds(off + rbase, RPS)])

        @pl.when(core0)
        def _():
            pltpu.sync_copy(hist, deg_h.at[s])

    return k(p2, src, dst)


def _lin_body(h_ref, w1_ref, b2_ref, out_ref):
    out_ref[...] = lax.dot_general(
        h_ref[...], w1_ref[...],
        (((1,), (1,)), ((), ())),
        preferred_element_type=jnp.float32,
    ) + b2_ref[...]


def _linear(h, W1, b2):
    return pl.pallas_call(
        _lin_body,
        grid=(_NRB,),
        in_specs=[
            pl.BlockSpec((_RB, D), lambda i: (i, 0)),
            pl.BlockSpec((D, D), lambda i: (0, 0)),
            pl.BlockSpec((1, D), lambda i: (0, 0)),
        ],
        out_specs=pl.BlockSpec((_RB, D), lambda i: (i, 0)),
        out_shape=jax.ShapeDtypeStruct((N, D), jnp.float32),
    )(h, W1, b2)


def _combine_body(q_ref, acc0_ref, acc1_ref, deg_ref, out_ref):
    q = q_ref[...]
    deg = jnp.sum(deg_ref[0], axis=0)[:, None]
    inv = 1.0 / jnp.maximum(deg, 1.0)
    hn = jnp.concatenate([acc0_ref[0], acc1_ref[0]], axis=1) * inv
    t = q + hn
    ss = jnp.sum(t * t, axis=1, keepdims=True)
    out_ref[...] = t / jnp.maximum(jnp.sqrt(ss), 1e-12)


def _combine(q, acc, deg):
    return pl.pallas_call(
        _combine_body,
        grid=(_NRB,),
        in_specs=[
            pl.BlockSpec((_RB, D), lambda i: (i, 0)),
            pl.BlockSpec((1, _RB, DH), lambda i: (0, i, 0)),
            pl.BlockSpec((1, _RB, DH), lambda i: (1, i, 0)),
            pl.BlockSpec((1, NS, _RB), lambda i: (i, 0, 0)),
        ],
        out_specs=pl.BlockSpec((_RB, D), lambda i: (i, 0)),
        out_shape=jax.ShapeDtypeStruct((N, D), jnp.float32),
    )(q, acc, acc, deg)


def kernel(h, edge_index, W1, W2, b2):
    src = edge_index[0]
    dst = edge_index[1]
    p2 = _proj(h, W2)
    acc, deg = _sc_agg(p2.reshape(NC * NPAD, DH), src, dst)
    q = _linear(h, W1, b2.reshape(1, D))
    deg3 = deg[:, :N].reshape(NS, _NRB, _RB).transpose(1, 0, 2)
    out = _combine(q, acc.reshape(NC, NPAD, DH), deg3)
    return out


# speedup vs baseline: 1.4175x; 1.1706x over previous
import dataclasses
import functools

import jax
import jax.numpy as jnp
from jax import lax
from jax.experimental import pallas as pl
from jax.experimental.pallas import tpu as pltpu
from jax.experimental.pallas import tpu_sc as plsc

N = 10000
E = 160000
D = 256
DH = 128
NPAD = 10240
NC = 2
NS = 16
EPS = E // NS
CH = 128
NCH = EPS // CH
TAIL = EPS - NCH * CH
ZR = 128
RPS = NPAD // NS


_RB = 2000
_NRB = N // _RB


def _proj_body(h_ref, w2_ref, out_ref):
    h_blk = h_ref[...]
    w2 = w2_ref[...]
    for half in range(NC):
        out_ref[half] = lax.dot_general(
            h_blk, w2[half * DH:(half + 1) * DH, :],
            (((1,), (1,)), ((), ())),
            preferred_element_type=jnp.float32,
        )


def _proj(h, W2):
    return pl.pallas_call(
        _proj_body,
        grid=(_NRB,),
        in_specs=[
            pl.BlockSpec((_RB, D), lambda i: (i, 0)),
            pl.BlockSpec((D, D), lambda i: (0, 0)),
        ],
        out_specs=pl.BlockSpec((NC, _RB, DH), lambda i: (0, i, 0)),
        out_shape=jax.ShapeDtypeStruct((NC, NPAD, DH), jnp.float32),
    )(h, W2)


def _sc_agg(p2, src, dst):
    mesh = plsc.VectorSubcoreMesh(core_axis_name="c", subcore_axis_name="s")
    cp = pltpu.CompilerParams()
    if "needs_layout_passes" in pltpu.CompilerParams.__dataclass_fields__:
        cp = dataclasses.replace(cp, needs_layout_passes=False)

    @functools.partial(
        pl.kernel,
        compiler_params=cp,
        out_type=(
            jax.ShapeDtypeStruct((NC * NPAD, DH), jnp.float32),
            jax.ShapeDtypeStruct((NS, NPAD), jnp.float32),
        ),
        mesh=mesh,
        scratch_types=[
            pltpu.VMEM((2, CH), jnp.int32),
            pltpu.VMEM((2, CH), jnp.int32),
            pltpu.VMEM((2, CH, DH), jnp.float32),
            pltpu.VMEM((TAIL,), jnp.int32),
            pltpu.VMEM((TAIL,), jnp.int32),
            pltpu.VMEM((TAIL, DH), jnp.float32),
            pltpu.VMEM((NPAD,), jnp.float32),
            pltpu.VMEM_SHARED((NPAD, DH), jnp.float32),
            pltpu.SemaphoreType.DMA,
            pltpu.SemaphoreType.DMA,
            pltpu.SemaphoreType.DMA,
            pltpu.SemaphoreType.DMA,
        ],
    )
    def k(p2_h, src_h, dst_h, acc_h, deg_h,
          isrc, idst, rows, tsrc, tdst, trows, hist,
          acc_sh, sg0, sg1, ss0, ss1):
        c = lax.axis_index("c")
        s = lax.axis_index("s")
        core0 = c == 0
        off = c * NPAD

        zv = jnp.zeros((16,), jnp.float32)
        ov = jnp.ones((16,), jnp.float32)

        @pl.loop(0, ZR)
        def _(i):
            @pl.loop(0, DH, step=16)
            def _(j):
                rows[0, i, pl.ds(j, 16)] = zv

        @pl.loop(0, NPAD, step=16)
        def _(i):
            hist[pl.ds(i, 16)] = zv

        rbase = s * RPS

        @pl.loop(0, RPS, step=ZR)
        def _(r):
            pltpu.sync_copy(rows.at[0], acc_sh.at[pl.ds(rbase + r, ZR)])

        plsc.subcore_barrier()

        ebase = s * EPS

        def load_idx(k_, b):
            pltpu.sync_copy(src_h.at[pl.ds(ebase + k_ * CH, CH)], isrc.at[b])
            pltpu.sync_copy(dst_h.at[pl.ds(ebase + k_ * CH, CH)], idst.at[b])

            @pl.loop(0, CH, step=16)
            def _(j):
                isrc[b, pl.ds(j, 16)] = isrc[b, pl.ds(j, 16)] + off

        ssem = (ss0, ss1)

        def scatter(b):
            d = pltpu.async_copy(rows.at[b], acc_sh.at[idst.at[b]], ssem[b],
                                 add=True)

            @pl.when(core0)
            def _():
                @pl.loop(0, CH, step=16)
                def _(j):
                    plsc.addupdate_scatter(hist, [idst[b, pl.ds(j, 16)]], ov)

            return d

        @pl.loop(0, NCH, step=2)
        def _(k_):
            load_idx(k_, 0)
            d0 = pltpu.async_copy(p2_h.at[isrc.at[0]], rows.at[0], sg0)
            load_idx(k_ + 1, 1)
            d1 = pltpu.async_copy(p2_h.at[isrc.at[1]], rows.at[1], sg1)
            d0.wait()
            s0 = scatter(0)
            d1.wait()
            s1 = scatter(1)
            s0.wait()
            s1.wait()

        tbase = ebase + NCH * CH
        pltpu.sync_copy(src_h.at[pl.ds(tbase, TAIL)], tsrc)
        pltpu.sync_copy(dst_h.at[pl.ds(tbase, TAIL)], tdst)
        tsrc[...] = tsrc[...] + off
        pltpu.sync_copy(p2_h.at[tsrc], trows)
        pltpu.sync_copy(trows, acc_sh.at[tdst], add=True)

        @pl.when(core0)
        def _():
            plsc.addupdate_scatter(hist, [tdst[...]], ov)

        plsc.subcore_barrier()

        pltpu.sync_copy(acc_sh.at[pl.ds(rbase, RPS)],
                        acc_h.at[pl.ds(off + rbase, RPS)])

        @pl.when(core0)
        def _():
            pltpu.sync_copy(hist, deg_h.at[s])

    return k(p2, src, dst)


def _lin_body(h_ref, w1_ref, b2_ref, out_ref):
    out_ref[...] = lax.dot_general(
        h_ref[...], w1_ref[...],
        (((1,), (1,)), ((), ())),
        preferred_element_type=jnp.float32,
    ) + b2_ref[...]


def _linear(h, W1, b2):
    return pl.pallas_call(
        _lin_body,
        grid=(_NRB,),
        in_specs=[
            pl.BlockSpec((_RB, D), lambda i: (i, 0)),
            pl.BlockSpec((D, D), lambda i: (0, 0)),
            pl.BlockSpec((1, D), lambda i: (0, 0)),
        ],
        out_specs=pl.BlockSpec((_RB, D), lambda i: (i, 0)),
        out_shape=jax.ShapeDtypeStruct((N, D), jnp.float32),
    )(h, W1, b2)


def _combine_body(q_ref, acc0_ref, acc1_ref, deg_ref, out_ref):
    q = q_ref[...]
    deg = jnp.sum(deg_ref[0], axis=0)[:, None]
    inv = 1.0 / jnp.maximum(deg, 1.0)
    hn = jnp.concatenate([acc0_ref[0], acc1_ref[0]], axis=1) * inv
    t = q + hn
    ss = jnp.sum(t * t, axis=1, keepdims=True)
    out_ref[...] = t / jnp.maximum(jnp.sqrt(ss), 1e-12)


def _combine(q, acc, deg):
    return pl.pallas_call(
        _combine_body,
        grid=(_NRB,),
        in_specs=[
            pl.BlockSpec((_RB, D), lambda i: (i, 0)),
            pl.BlockSpec((1, _RB, DH), lambda i: (0, i, 0)),
            pl.BlockSpec((1, _RB, DH), lambda i: (1, i, 0)),
            pl.BlockSpec((1, NS, _RB), lambda i: (i, 0, 0)),
        ],
        out_specs=pl.BlockSpec((_RB, D), lambda i: (i, 0)),
        out_shape=jax.ShapeDtypeStruct((N, D), jnp.float32),
    )(q, acc, acc, deg)


def kernel(h, edge_index, W1, W2, b2):
    src = edge_index[0]
    dst = edge_index[1]
    p2 = _proj(h, W2)
    acc, deg = _sc_agg(p2.reshape(NC * NPAD, DH), src, dst)
    q = _linear(h, W1, b2.reshape(1, D))
    deg3 = deg[:, :N].reshape(NS, _NRB, _RB).transpose(1, 0, 2)
    out = _combine(q, acc.reshape(NC, NPAD, DH), deg3)
    return out


# speedup vs baseline: 1.4478x; 1.0213x over previous
import dataclasses
import functools

import jax
import jax.numpy as jnp
from jax import lax
from jax.experimental import pallas as pl
from jax.experimental.pallas import tpu as pltpu
from jax.experimental.pallas import tpu_sc as plsc

N = 10000
E = 160000
D = 256
DH = 128
NPAD = 10240
NC = 2
NS = 16
EPS = E // NS
CH = 128
NCH = EPS // CH
TAIL = EPS - NCH * CH
ZR = 128
RPS = NPAD // NS


_RB = 2000
_NRB = N // _RB


def _proj_body(h_ref, w2_ref, out_ref):
    h_blk = h_ref[...]
    w2 = w2_ref[...]
    for half in range(NC):
        out_ref[half] = lax.dot_general(
            h_blk, w2[half * DH:(half + 1) * DH, :],
            (((1,), (1,)), ((), ())),
            preferred_element_type=jnp.float32,
        )


def _proj(h, W2):
    return pl.pallas_call(
        _proj_body,
        grid=(_NRB,),
        in_specs=[
            pl.BlockSpec((_RB, D), lambda i: (i, 0)),
            pl.BlockSpec((D, D), lambda i: (0, 0)),
        ],
        out_specs=pl.BlockSpec((NC, _RB, DH), lambda i: (0, i, 0)),
        out_shape=jax.ShapeDtypeStruct((NC, NPAD, DH), jnp.float32),
    )(h, W2)


def _sc_agg(p2, ei):
    mesh = plsc.VectorSubcoreMesh(core_axis_name="c", subcore_axis_name="s")
    cp = pltpu.CompilerParams()
    if "needs_layout_passes" in pltpu.CompilerParams.__dataclass_fields__:
        cp = dataclasses.replace(cp, needs_layout_passes=False)

    @functools.partial(
        pl.kernel,
        compiler_params=cp,
        out_type=(
            jax.ShapeDtypeStruct((NC * NPAD, DH), jnp.float32),
            jax.ShapeDtypeStruct((NS, NPAD), jnp.float32),
        ),
        mesh=mesh,
        scratch_types=[
            pltpu.VMEM((2, CH), jnp.int32),
            pltpu.VMEM((2, CH), jnp.int32),
            pltpu.VMEM((2, CH, DH), jnp.float32),
            pltpu.VMEM((TAIL,), jnp.int32),
            pltpu.VMEM((TAIL,), jnp.int32),
            pltpu.VMEM((TAIL, DH), jnp.float32),
            pltpu.VMEM((NPAD,), jnp.float32),
            pltpu.VMEM_SHARED((NPAD, DH), jnp.float32),
            pltpu.SemaphoreType.DMA,
            pltpu.SemaphoreType.DMA,
            pltpu.SemaphoreType.DMA,
            pltpu.SemaphoreType.DMA,
        ],
    )
    def k(p2_h, ei_h, acc_h, deg_h,
          isrc, idst, rows, tsrc, tdst, trows, hist,
          acc_sh, sg0, sg1, ss0, ss1):
        src_h = ei_h.at[pl.ds(0, E)]
        dst_h = ei_h.at[pl.ds(E, E)]
        c = lax.axis_index("c")
        s = lax.axis_index("s")
        core0 = c == 0
        off = c * NPAD

        zv = jnp.zeros((16,), jnp.float32)
        ov = jnp.ones((16,), jnp.float32)

        @pl.loop(0, ZR)
        def _(i):
            @pl.loop(0, DH, step=16)
            def _(j):
                rows[0, i, pl.ds(j, 16)] = zv

        @pl.loop(0, NPAD, step=16)
        def _(i):
            hist[pl.ds(i, 16)] = zv

        rbase = s * RPS

        @pl.loop(0, RPS, step=ZR)
        def _(r):
            pltpu.sync_copy(rows.at[0], acc_sh.at[pl.ds(rbase + r, ZR)])

        plsc.subcore_barrier()

        ebase = s * EPS

        def load_idx(k_, b):
            pltpu.sync_copy(src_h.at[pl.ds(ebase + k_ * CH, CH)], isrc.at[b])
            pltpu.sync_copy(dst_h.at[pl.ds(ebase + k_ * CH, CH)], idst.at[b])

            @pl.loop(0, CH, step=16)
            def _(j):
                isrc[b, pl.ds(j, 16)] = isrc[b, pl.ds(j, 16)] + off

        ssem = (ss0, ss1)

        def scatter(b):
            d = pltpu.async_copy(rows.at[b], acc_sh.at[idst.at[b]], ssem[b],
                                 add=True)

            @pl.when(core0)
            def _():
                @pl.loop(0, CH, step=16)
                def _(j):
                    plsc.addupdate_scatter(hist, [idst[b, pl.ds(j, 16)]], ov)

            return d

        @pl.loop(0, NCH, step=2)
        def _(k_):
            load_idx(k_, 0)
            d0 = pltpu.async_copy(p2_h.at[isrc.at[0]], rows.at[0], sg0)
            load_idx(k_ + 1, 1)
            d1 = pltpu.async_copy(p2_h.at[isrc.at[1]], rows.at[1], sg1)
            d0.wait()
            s0 = scatter(0)
            d1.wait()
            s1 = scatter(1)
            s0.wait()
            s1.wait()

        tbase = ebase + NCH * CH
        pltpu.sync_copy(src_h.at[pl.ds(tbase, TAIL)], tsrc)
        pltpu.sync_copy(dst_h.at[pl.ds(tbase, TAIL)], tdst)
        tsrc[...] = tsrc[...] + off
        pltpu.sync_copy(p2_h.at[tsrc], trows)
        pltpu.sync_copy(trows, acc_sh.at[tdst], add=True)

        @pl.when(core0)
        def _():
            plsc.addupdate_scatter(hist, [tdst[...]], ov)

        plsc.subcore_barrier()

        pltpu.sync_copy(acc_sh.at[pl.ds(rbase, RPS)],
                        acc_h.at[pl.ds(off + rbase, RPS)])

        @pl.when(core0)
        def _():
            pltpu.sync_copy(hist, deg_h.at[s])

    return k(p2, ei)


def _lin_body(h_ref, w1_ref, b2_ref, out_ref):
    out_ref[...] = lax.dot_general(
        h_ref[...], w1_ref[...],
        (((1,), (1,)), ((), ())),
        preferred_element_type=jnp.float32,
    ) + b2_ref[...]


def _linear(h, W1, b2):
    return pl.pallas_call(
        _lin_body,
        grid=(_NRB,),
        in_specs=[
            pl.BlockSpec((_RB, D), lambda i: (i, 0)),
            pl.BlockSpec((D, D), lambda i: (0, 0)),
            pl.BlockSpec((1, D), lambda i: (0, 0)),
        ],
        out_specs=pl.BlockSpec((_RB, D), lambda i: (i, 0)),
        out_shape=jax.ShapeDtypeStruct((N, D), jnp.float32),
    )(h, W1, b2)


def _combine_body(q_ref, acc0_ref, acc1_ref, deg_ref, out_ref):
    q = q_ref[...]
    deg = jnp.sum(deg_ref[0], axis=0)[:, None]
    inv = 1.0 / jnp.maximum(deg, 1.0)
    hn = jnp.concatenate([acc0_ref[0], acc1_ref[0]], axis=1) * inv
    t = q + hn
    ss = jnp.sum(t * t, axis=1, keepdims=True)
    out_ref[...] = t / jnp.maximum(jnp.sqrt(ss), 1e-12)


def _combine(q, acc, deg):
    return pl.pallas_call(
        _combine_body,
        grid=(_NRB,),
        in_specs=[
            pl.BlockSpec((_RB, D), lambda i: (i, 0)),
            pl.BlockSpec((1, _RB, DH), lambda i: (0, i, 0)),
            pl.BlockSpec((1, _RB, DH), lambda i: (1, i, 0)),
            pl.BlockSpec((1, NS, _RB), lambda i: (i, 0, 0)),
        ],
        out_specs=pl.BlockSpec((_RB, D), lambda i: (i, 0)),
        out_shape=jax.ShapeDtypeStruct((N, D), jnp.float32),
    )(q, acc, acc, deg)


def kernel(h, edge_index, W1, W2, b2):
    p2 = _proj(h, W2)
    acc, deg = _sc_agg(p2.reshape(NC * NPAD, DH), edge_index.reshape(2 * E))
    q = _linear(h, W1, b2.reshape(1, D))
    deg3 = deg[:, :N].reshape(NS, _NRB, _RB).transpose(1, 0, 2)
    out = _combine(q, acc.reshape(NC, NPAD, DH), deg3)
    return out


# speedup vs baseline: 1.5226x; 1.0517x over previous
import dataclasses
import functools

import jax
import jax.numpy as jnp
from jax import lax
from jax.experimental import pallas as pl
from jax.experimental.pallas import tpu as pltpu
from jax.experimental.pallas import tpu_sc as plsc

N = 10000
E = 160000
D = 256
DH = 128
NPAD = 10240
NC = 2
NS = 16
CH = 128
NROW = E // CH
NSLAB = 10
XSUB = 156 - (NSLAB - 1) * NS
ZR = 128
RPS = NPAD // NS


_RB = 2000
_NRB = N // _RB


def _proj_body(h_ref, w2_ref, out_ref):
    h_blk = h_ref[...]
    w2 = w2_ref[...]
    for half in range(NC):
        out_ref[half] = lax.dot_general(
            h_blk, w2[half * DH:(half + 1) * DH, :],
            (((1,), (1,)), ((), ())),
            preferred_element_type=jnp.float32,
        )


def _proj(h, W2):
    return pl.pallas_call(
        _proj_body,
        grid=(_NRB,),
        in_specs=[
            pl.BlockSpec((_RB, D), lambda i: (i, 0)),
            pl.BlockSpec((D, D), lambda i: (0, 0)),
        ],
        out_specs=pl.BlockSpec((NC, _RB, DH), lambda i: (0, i, 0)),
        out_shape=jax.ShapeDtypeStruct((NC, NPAD, DH), jnp.float32),
    )(h, W2)


def _sc_agg(p2, ei3):
    mesh = plsc.VectorSubcoreMesh(core_axis_name="c", subcore_axis_name="s")
    cp = pltpu.CompilerParams()
    if "needs_layout_passes" in pltpu.CompilerParams.__dataclass_fields__:
        cp = dataclasses.replace(cp, needs_layout_passes=False)

    @functools.partial(
        pl.kernel,
        compiler_params=cp,
        out_type=(
            jax.ShapeDtypeStruct((NC * NPAD, DH), jnp.float32),
            jax.ShapeDtypeStruct((NS, NPAD), jnp.float32),
        ),
        mesh=mesh,
        scratch_types=[
            pltpu.VMEM((8, CH), jnp.int32),
            pltpu.VMEM((8, CH), jnp.int32),
            pltpu.VMEM((2, CH, DH), jnp.float32),
            pltpu.VMEM((NPAD,), jnp.float32),
            pltpu.VMEM_SHARED((NPAD, DH), jnp.float32),
            pltpu.SemaphoreType.DMA,
            pltpu.SemaphoreType.DMA,
            pltpu.SemaphoreType.DMA,
            pltpu.SemaphoreType.DMA,
        ],
    )
    def k(p2_h, ei_h, acc_h, deg_h,
          isrc, idst, rows, hist,
          acc_sh, sg0, sg1, ss0, ss1):
        src_h = ei_h.at[0]
        dst_h = ei_h.at[1]
        c = lax.axis_index("c")
        s = lax.axis_index("s")
        core0 = c == 0
        off = c * NPAD

        zv = jnp.zeros((16,), jnp.float32)
        ov = jnp.ones((16,), jnp.float32)

        @pl.loop(0, ZR)
        def _(i):
            @pl.loop(0, DH, step=16)
            def _(j):
                rows[0, i, pl.ds(j, 16)] = zv

        @pl.loop(0, NPAD, step=16)
        def _(i):
            hist[pl.ds(i, 16)] = zv

        rbase = s * RPS

        @pl.loop(0, RPS, step=ZR)
        def _(r):
            pltpu.sync_copy(rows.at[0], acc_sh.at[pl.ds(rbase + r, ZR)])

        plsc.subcore_barrier()

        sgs = (sg0, sg1)
        ssem = (ss0, ss1)

        def load_slab(r_, n):
            pltpu.sync_copy(src_h.at[pl.ds(r_, n)], isrc.at[pl.ds(0, n)])
            pltpu.sync_copy(dst_h.at[pl.ds(r_, n)], idst.at[pl.ds(0, n)])
            for b in range(n):
                @pl.loop(0, CH, step=16)
                def _(j, b=b):
                    isrc[b, pl.ds(j, 16)] = isrc[b, pl.ds(j, 16)] + off

        def gather(b, rb):
            return pltpu.async_copy(p2_h.at[isrc.at[b]], rows.at[rb], sgs[rb])

        def scatter(b, rb):
            d = pltpu.async_copy(rows.at[rb], acc_sh.at[idst.at[b]],
                                 ssem[rb], add=True)

            @pl.when(core0)
            def _():
                @pl.loop(0, CH, step=16)
                def _(j):
                    plsc.addupdate_scatter(hist, [idst[b, pl.ds(j, 16)]], ov)

            return d

        def quad(b0):
            d0 = gather(b0, 0)
            d1 = gather(b0 + 1, 1)
            d0.wait()
            s0 = scatter(b0, 0)
            d1.wait()
            s1 = scatter(b0 + 1, 1)
            s0.wait()
            d0 = gather(b0 + 2, 0)
            s1.wait()
            d1 = gather(b0 + 3, 1)
            d0.wait()
            s0 = scatter(b0 + 2, 0)
            d1.wait()
            s1 = scatter(b0 + 3, 1)
            s0.wait()
            s1.wait()

        def slab8(r_):
            load_slab(r_, 8)
            quad(0)
            quad(4)

        @pl.loop(0, NSLAB - 1)
        def _(i):
            slab8(i * NS * 8 + s * 8)

        @pl.when(s < XSUB)
        def _():
            slab8((NSLAB - 1) * NS * 8 + s * 8)

        @pl.when(s == 0)
        def _():
            load_slab(NROW - 2, 2)
            d0 = gather(0, 0)
            d1 = gather(1, 1)
            d0.wait()
            s0 = scatter(0, 0)
            d1.wait()
            s1 = scatter(1, 1)
            s0.wait()
            s1.wait()

        plsc.subcore_barrier()

        pltpu.sync_copy(acc_sh.at[pl.ds(rbase, RPS)],
                        acc_h.at[pl.ds(off + rbase, RPS)])

        @pl.when(core0)
        def _():
            pltpu.sync_copy(hist, deg_h.at[s])

    return k(p2, ei3)


def _lin_body(h_ref, w1_ref, b2_ref, out_ref):
    out_ref[...] = lax.dot_general(
        h_ref[...], w1_ref[...],
        (((1,), (1,)), ((), ())),
        preferred_element_type=jnp.float32,
    ) + b2_ref[...]


def _linear(h, W1, b2):
    return pl.pallas_call(
        _lin_body,
        grid=(_NRB,),
        in_specs=[
            pl.BlockSpec((_RB, D), lambda i: (i, 0)),
            pl.BlockSpec((D, D), lambda i: (0, 0)),
            pl.BlockSpec((1, D), lambda i: (0, 0)),
        ],
        out_specs=pl.BlockSpec((_RB, D), lambda i: (i, 0)),
        out_shape=jax.ShapeDtypeStruct((N, D), jnp.float32),
    )(h, W1, b2)


def _combine_body(q_ref, acc0_ref, acc1_ref, deg_ref, out_ref):
    q = q_ref[...]
    deg = jnp.sum(deg_ref[0], axis=0)[:, None]
    inv = 1.0 / jnp.maximum(deg, 1.0)
    hn = jnp.concatenate([acc0_ref[0], acc1_ref[0]], axis=1) * inv
    t = q + hn
    ss = jnp.sum(t * t, axis=1, keepdims=True)
    out_ref[...] = t / jnp.maximum(jnp.sqrt(ss), 1e-12)


def _combine(q, acc, deg):
    return pl.pallas_call(
        _combine_body,
        grid=(_NRB,),
        in_specs=[
            pl.BlockSpec((_RB, D), lambda i: (i, 0)),
            pl.BlockSpec((1, _RB, DH), lambda i: (0, i, 0)),
            pl.BlockSpec((1, _RB, DH), lambda i: (1, i, 0)),
            pl.BlockSpec((1, NS, _RB), lambda i: (i, 0, 0)),
        ],
        out_specs=pl.BlockSpec((_RB, D), lambda i: (i, 0)),
        out_shape=jax.ShapeDtypeStruct((N, D), jnp.float32),
    )(q, acc, acc, deg)


def kernel(h, edge_index, W1, W2, b2):
    p2 = _proj(h, W2)
    acc, deg = _sc_agg(p2.reshape(NC * NPAD, DH),
                       edge_index.reshape(2, NROW, CH))
    q = _linear(h, W1, b2.reshape(1, D))
    deg3 = deg[:, :N].reshape(NS, _NRB, _RB).transpose(1, 0, 2)
    out = _combine(q, acc.reshape(NC, NPAD, DH), deg3)
    return out
